# baseline (speedup 1.0000x reference)
import jax
import jax.numpy as jnp
from jax.experimental import pallas as pl
from jax.experimental.pallas import tpu as pltpu

_BN = 1000


def _gat_reduce_kernel(a_ref, ft_ref, o_ref):
    ablk = a_ref[...]
    e = jax.nn.softmax(ablk[None, :, :], axis=0)[0]
    acc = jnp.sum(ft_ref[...], axis=0)
    o_ref[...] = acc * e


def kernel(a, ft):
    deg, n, d = ft.shape
    grid = (n // _BN,)
    return pl.pallas_call(
        _gat_reduce_kernel,
        grid=grid,
        in_specs=[
            pl.BlockSpec((_BN, 1), lambda i: (i, 0)),
            pl.BlockSpec((deg, _BN, d), lambda i: (0, i, 0)),
        ],
        out_specs=pl.BlockSpec((_BN, d), lambda i: (i, 0)),
        out_shape=jax.ShapeDtypeStruct((n, d), ft.dtype),
        compiler_params=pltpu.CompilerParams(
            dimension_semantics=("arbitrary",),
        ),
    )(a, ft)
